# baseline (speedup 1.0000x reference)
import functools

import jax
import jax.numpy as jnp
from jax import lax
from jax.experimental import pallas as pl
from jax.experimental.pallas import tpu as pltpu
from jax.experimental.pallas import tpu_sc as plsc

L_SPH = 3
K = (L_SPH + 1) ** 2
NUM_CLASS = 20
STRIDE = [4, 3, 3, 2, 2]
ENC = [32, 64, 96, 128, 192, 256]
DEC = [256, 128, 128, 96, 96, 96]
NITERS = [2, 2, 4, 4, 4]
GROWTH = 32


def _norm(v):
    return jnp.sqrt(jnp.sum(v * v, axis=-1, keepdims=True) + 1e-12)


def _sph_harm(n):
    x, y, z = n[:, 0], n[:, 1], n[:, 2]
    one = jnp.ones_like(x)
    c = [0.28209479 * one, 0.48860251 * y, 0.48860251 * z, 0.48860251 * x,
         1.09254843 * x * y, 1.09254843 * y * z, 0.31539157 * (3.0 * z * z - 1.0),
         1.09254843 * x * z, 0.54627422 * (x * x - y * y),
         0.59004359 * y * (3.0 * x * x - y * y), 2.89061144 * x * y * z,
         0.45704580 * y * (5.0 * z * z - 1.0), 0.37317633 * z * (5.0 * z * z - 3.0),
         0.45704580 * x * (5.0 * z * z - 1.0), 1.44530572 * z * (x * x - y * y),
         0.59004359 * x * (x * x - 3.0 * y * y)]
    return jnp.stack(c, axis=1)


def _seg_mean_pos(v, s):
    nv = v.shape[0]
    nvo = max(nv // s, 1)
    main = v[: nvo * s].reshape(nvo, s, v.shape[1]).sum(axis=1)
    tail = v[nvo * s:]
    cnt = jnp.full((nvo, 1), float(s), jnp.float32)
    if tail.shape[0]:
        main = main.at[-1].add(tail.sum(axis=0))
        cnt = cnt.at[-1].add(float(tail.shape[0]))
    return main / cnt


def _seg_max(x, s):
    nv = x.shape[0]
    nvo = max(nv // s, 1)
    main = x[: nvo * s].reshape(nvo, s, x.shape[1]).max(axis=1)
    tail = x[nvo * s:]
    if tail.shape[0]:
        main = main.at[-1].max(tail.max(axis=0))
    return main


def _upsample(x, s, nv):
    nvo = x.shape[0]
    up = jnp.repeat(x, s, axis=0)
    if nv > nvo * s:
        up = jnp.concatenate(
            [up, jnp.broadcast_to(x[-1:], (nv - nvo * s, x.shape[1]))], axis=0)
    return up[:nv]


def _oriented_normals(V1, V2, V3):
    e1 = V2 - V1
    e2 = V3 - V1
    n = jnp.cross(e1, e2)
    n = n / _norm(n)
    center = (V1 + V2 + V3) / 3.0
    sign = jnp.sign(jnp.sum(-center * n, axis=-1, keepdims=True))
    sign = jnp.where(sign > -1.0, 1.0, sign)
    return n * sign


def _facet_geometry(V1, V2, V3, normals):
    D12 = V2 - V1
    D23 = V3 - V2
    D31 = V1 - V3
    L12 = _norm(D12)
    L23 = _norm(D23)
    L31 = _norm(D31)
    T1 = jnp.sum(D12 * (-D31), axis=-1, keepdims=True) / (L12 * L31)
    T2 = jnp.sum((-D12) * D23, axis=-1, keepdims=True) / (L12 * L23)
    T3 = jnp.sum((-D23) * D31, axis=-1, keepdims=True) / (L23 * L31)
    return jnp.concatenate([L12, L23, L31, T1, T2, T3, normals], axis=-1)


def _count_adj(face, nv):
    c = jnp.zeros((nv,), jnp.float32)
    return c.at[face[:, 0]].add(1.0).at[face[:, 1]].add(1.0).at[face[:, 2]].add(1.0)


def _spharm_conv(ff, filt, W):
    return jnp.einsum('nk,nc,kcg->ng', filt, ff, W)

def _spconv_body(ff_ref, filt_ref, w_ref, b_ref, o_ref):
    acc = jnp.broadcast_to(b_ref[...], o_ref.shape).astype(jnp.float32)
    ff = ff_ref[...]
    filt = filt_ref[...]
    for k in range(K):
        acc = acc + filt[:, k:k + 1] * jnp.dot(ff, w_ref[k],
                                               preferred_element_type=jnp.float32)
    o_ref[...] = acc


def _spharm_conv_b(ff, filt, W, b, block=2048):
    m, cin = ff.shape
    g = W.shape[2]
    grid = (pl.cdiv(m, block),)
    return pl.pallas_call(
        _spconv_body,
        grid=grid,
        in_specs=[
            pl.BlockSpec((block, cin), lambda i: (i, 0)),
            pl.BlockSpec((block, K), lambda i: (i, 0)),
            pl.BlockSpec((K, cin, g), lambda i: (0, 0, 0)),
            pl.BlockSpec((g,), lambda i: (0,)),
        ],
        out_specs=pl.BlockSpec((block, g), lambda i: (i, 0)),
        out_shape=jax.ShapeDtypeStruct((m, g), jnp.float32),
    )(ff, filt, W, b)


def _v2f(v_feats, face):
    return (v_feats[face[:, 0]] + v_feats[face[:, 1]] + v_feats[face[:, 2]]) / 3.0


def _f2v(f_feats, face, nv, nf_count):
    out = jnp.zeros((nv, f_feats.shape[1]), jnp.float32)
    out = out.at[face[:, 0]].add(f_feats).at[face[:, 1]].add(f_feats).at[face[:, 2]].add(f_feats)
    return out / jnp.maximum(nf_count[:, None], 1.0)


def _mesh_conv(v_feats, face, nv, nf_count, filt, W, b):
    ff = _v2f(v_feats, face)
    out = _spharm_conv_b(ff, filt, W, b)
    return jax.nn.relu(_f2v(out, face, nv, nf_count))


@functools.lru_cache(maxsize=1)
def _sc_mesh():
    return plsc.VectorSubcoreMesh(core_axis_name="c", subcore_axis_name="s")


def _gather_layout(e):
    cands = []
    for active in range(1, 17):
        gran = 2 * active * 128
        trips = -(-e // gran)
        cands.append((gran * trips, active, trips))
    mp_min = min(c[0] for c in cands)
    ep, active, trips = max((c for c in cands if c[0] <= mp_min * 1.10),
                            key=lambda c: c[1])
    return active, trips, ep


def _sc_gather(table, idx, active, trips):
    t_rows, cc = table.shape
    ep = idx.shape[0]

    def body(table_hbm, idx_hbm, out_hbm, idx_v, rows_v, sem):
        c = lax.axis_index("c")
        s = lax.axis_index("s")
        wid = s * 2 + c
        base = wid * (trips * 128)

        def chunk(t, carry):
            off = pl.multiple_of(base + t * 128, 8)
            pltpu.sync_copy(idx_hbm.at[pl.ds(off, 128)], idx_v)
            pltpu.async_copy(table_hbm.at[idx_v], rows_v, sem).wait()
            pltpu.sync_copy(rows_v, out_hbm.at[pl.ds(off, 128)])
            return carry

        @pl.when(wid < 2 * active)
        def _():
            lax.fori_loop(0, trips, chunk, 0)

    return pl.kernel(
        body,
        out_type=jax.ShapeDtypeStruct((ep, cc), jnp.float32),
        mesh=_sc_mesh(),
        compiler_params=pltpu.CompilerParams(use_tc_tiling_on_sc=False),
        scratch_types=[
            pltpu.VMEM((128,), jnp.int32),
            pltpu.VMEM((128, cc), jnp.float32),
            pltpu.SemaphoreType.DMA,
        ],
    )(table, idx)


def _pad_corner_idx(face, ep):
    m = face.shape[0]
    flat = face.T.reshape(3 * m).astype(jnp.int32)
    return jnp.zeros((ep,), jnp.int32).at[:3 * m].set(flat)


def _v2f_sc(v_feats, face):
    m = face.shape[0]
    active, trips, ep = _gather_layout(3 * m)
    idx = _pad_corner_idx(face, ep)
    rows = _sc_gather(v_feats, idx, active, trips)
    g = rows[:3 * m].reshape(3, m, v_feats.shape[1])
    return (g[0] + g[1] + g[2]) * (1.0 / 3.0)


def _corner_gather_sc(v_pad16, face):
    m = face.shape[0]
    active, trips, ep = _gather_layout(3 * m)
    idx = _pad_corner_idx(face, ep)
    rows = _sc_gather(v_pad16, idx, active, trips)
    return rows[:3 * m].reshape(3, m, 16)


def _matmul_kernel(x_ref, w_ref, b_ref, o_ref):
    o_ref[...] = jnp.dot(x_ref[...], w_ref[...],
                         preferred_element_type=jnp.float32) + b_ref[...]


def _matmul_relu_kernel(x_ref, w_ref, b_ref, o_ref):
    o_ref[...] = jax.nn.relu(jnp.dot(x_ref[...], w_ref[...],
                                     preferred_element_type=jnp.float32) + b_ref[...])


def _matmul_bias(x, W, b, block=1024, relu=False):
    n, cin = x.shape
    cout = W.shape[1]
    grid = (pl.cdiv(n, block),)
    return pl.pallas_call(
        _matmul_relu_kernel if relu else _matmul_kernel,
        grid=grid,
        in_specs=[
            pl.BlockSpec((block, cin), lambda i: (i, 0)),
            pl.BlockSpec((cin, cout), lambda i: (0, 0)),
            pl.BlockSpec((cout,), lambda i: (0,)),
        ],
        out_specs=pl.BlockSpec((block, cout), lambda i: (i, 0)),
        out_shape=jax.ShapeDtypeStruct((n, cout), jnp.float32),
    )(x, W, b)


def kernel(vertex_in, face_in, nv_in, mf_in, facet_textures, bary_coeff, num_texture, params):
    vertex = vertex_in[:, :3]
    nv0 = vertex.shape[0]

    vs = [vertex]
    fs = [face_in]
    nvs = [nv0]
    for s in STRIDE:
        nv = nvs[-1]
        nvo = max(nv // s, 1)
        vs.append(_seg_mean_pos(vs[-1], s))
        f_next = jnp.minimum(fs[-1] // s, nvo - 1)[::s]
        fs.append(f_next)
        nvs.append(nvo)

    f0 = fs[0]
    V1, V2, V3 = vertex[f0[:, 0]], vertex[f0[:, 1]], vertex[f0[:, 2]]
    n0 = _oriented_normals(V1, V2, V3)
    geom = _facet_geometry(V1, V2, V3, n0)
    filt0 = _sph_harm(n0)
    geo_face = jax.nn.relu(geom @ params['Wg'] + params['bg'])
    bary_sph = _sph_harm(bary_coeff)
    tex_face = _spharm_conv_b(facet_textures, bary_sph, params['Wt'],
                              jnp.zeros((ENC[0],), jnp.float32))
    denom = jnp.maximum(num_texture.astype(jnp.float32)[:, None], 1.0)
    tex_face = jax.nn.relu(tex_face / denom + params['bt0'])
    face_feat = geo_face + tex_face

    nfc0 = _count_adj(f0, nv0)
    feats = jax.nn.relu(_f2v(_spharm_conv_b(face_feat, filt0, params['Wf0'],
                                            params['bf0']), f0, nv0, nfc0))

    decoder_helper = [feats]
    feats = _seg_max(feats, STRIDE[0])

    for k in range(5):
        fk = fs[k + 1]
        nvk = nvs[k + 1]
        nfck = _count_adj(fk, nvk)
        Vk = vs[k + 1]
        nk = _oriented_normals(Vk[fk[:, 0]], Vk[fk[:, 1]], Vk[fk[:, 2]])
        filtk = _sph_harm(nk)
        for i in range(NITERS[k]):
            new = _mesh_conv(feats, fk, nvk, nfck, filtk,
                             params['B%d_W%d' % (k, i)], params['B%d_b%d' % (k, i)])
            feats = jnp.concatenate([feats, new], axis=-1)
        feats = _matmul_bias(feats, params['B%d_Wt' % k], params['B%d_bt' % k], relu=True)
        if k < 4:
            decoder_helper.append(feats)
            feats = _seg_max(feats, STRIDE[k + 1])

    for k in range(5):
        it = 5 - k
        low = decoder_helper[it - 1]
        up = _upsample(feats, STRIDE[it - 1], nvs[it - 1])
        feats = jnp.concatenate([low, up], axis=-1)
        feats = _matmul_bias(feats, params['D%d_W' % k], params['D%d_b' % k], relu=True)

    return _matmul_bias(feats, params['Wp'], params['bp'])


# speedup vs baseline: 1.4117x; 1.4117x over previous
import functools

import jax
import jax.numpy as jnp
from jax import lax
from jax.experimental import pallas as pl
from jax.experimental.pallas import tpu as pltpu
from jax.experimental.pallas import tpu_sc as plsc

L_SPH = 3
K = (L_SPH + 1) ** 2
NUM_CLASS = 20
STRIDE = [4, 3, 3, 2, 2]
ENC = [32, 64, 96, 128, 192, 256]
DEC = [256, 128, 128, 96, 96, 96]
NITERS = [2, 2, 4, 4, 4]
GROWTH = 32


def _norm(v):
    return jnp.sqrt(jnp.sum(v * v, axis=-1, keepdims=True) + 1e-12)


def _sph_harm(n):
    x, y, z = n[:, 0], n[:, 1], n[:, 2]
    one = jnp.ones_like(x)
    c = [0.28209479 * one, 0.48860251 * y, 0.48860251 * z, 0.48860251 * x,
         1.09254843 * x * y, 1.09254843 * y * z, 0.31539157 * (3.0 * z * z - 1.0),
         1.09254843 * x * z, 0.54627422 * (x * x - y * y),
         0.59004359 * y * (3.0 * x * x - y * y), 2.89061144 * x * y * z,
         0.45704580 * y * (5.0 * z * z - 1.0), 0.37317633 * z * (5.0 * z * z - 3.0),
         0.45704580 * x * (5.0 * z * z - 1.0), 1.44530572 * z * (x * x - y * y),
         0.59004359 * x * (x * x - 3.0 * y * y)]
    return jnp.stack(c, axis=1)


def _seg_mean_pos(v, s):
    nv = v.shape[0]
    nvo = max(nv // s, 1)
    main = v[: nvo * s].reshape(nvo, s, v.shape[1]).sum(axis=1)
    tail = v[nvo * s:]
    cnt = jnp.full((nvo, 1), float(s), jnp.float32)
    if tail.shape[0]:
        main = main.at[-1].add(tail.sum(axis=0))
        cnt = cnt.at[-1].add(float(tail.shape[0]))
    return main / cnt


def _seg_max(x, s):
    nv = x.shape[0]
    nvo = max(nv // s, 1)
    main = x[: nvo * s].reshape(nvo, s, x.shape[1]).max(axis=1)
    tail = x[nvo * s:]
    if tail.shape[0]:
        main = main.at[-1].max(tail.max(axis=0))
    return main


def _upsample(x, s, nv):
    nvo = x.shape[0]
    up = jnp.repeat(x, s, axis=0)
    if nv > nvo * s:
        up = jnp.concatenate(
            [up, jnp.broadcast_to(x[-1:], (nv - nvo * s, x.shape[1]))], axis=0)
    return up[:nv]


def _oriented_normals(V1, V2, V3):
    e1 = V2 - V1
    e2 = V3 - V1
    n = jnp.cross(e1, e2)
    n = n / _norm(n)
    center = (V1 + V2 + V3) / 3.0
    sign = jnp.sign(jnp.sum(-center * n, axis=-1, keepdims=True))
    sign = jnp.where(sign > -1.0, 1.0, sign)
    return n * sign


def _facet_geometry(V1, V2, V3, normals):
    D12 = V2 - V1
    D23 = V3 - V2
    D31 = V1 - V3
    L12 = _norm(D12)
    L23 = _norm(D23)
    L31 = _norm(D31)
    T1 = jnp.sum(D12 * (-D31), axis=-1, keepdims=True) / (L12 * L31)
    T2 = jnp.sum((-D12) * D23, axis=-1, keepdims=True) / (L12 * L23)
    T3 = jnp.sum((-D23) * D31, axis=-1, keepdims=True) / (L23 * L31)
    return jnp.concatenate([L12, L23, L31, T1, T2, T3, normals], axis=-1)


def _count_adj(face, nv):
    c = jnp.zeros((nv,), jnp.float32)
    return c.at[face[:, 0]].add(1.0).at[face[:, 1]].add(1.0).at[face[:, 2]].add(1.0)


def _spharm_conv(ff, filt, W):
    return jnp.einsum('nk,nc,kcg->ng', filt, ff, W)

def _spconv_body(ff_ref, filt_ref, w_ref, b_ref, o_ref):
    acc = jnp.broadcast_to(b_ref[...], o_ref.shape).astype(jnp.float32)
    ff = ff_ref[...]
    filt = filt_ref[...]
    for k in range(K):
        acc = acc + filt[:, k:k + 1] * jnp.dot(ff, w_ref[k],
                                               preferred_element_type=jnp.float32)
    o_ref[...] = acc


def _spharm_conv_b(ff, filt, W, b, block=2048):
    m, cin = ff.shape
    g = W.shape[2]
    grid = (pl.cdiv(m, block),)
    return pl.pallas_call(
        _spconv_body,
        grid=grid,
        in_specs=[
            pl.BlockSpec((block, cin), lambda i: (i, 0)),
            pl.BlockSpec((block, K), lambda i: (i, 0)),
            pl.BlockSpec((K, cin, g), lambda i: (0, 0, 0)),
            pl.BlockSpec((g,), lambda i: (0,)),
        ],
        out_specs=pl.BlockSpec((block, g), lambda i: (i, 0)),
        out_shape=jax.ShapeDtypeStruct((m, g), jnp.float32),
    )(ff, filt, W, b)


def _v2f(v_feats, face):
    return (v_feats[face[:, 0]] + v_feats[face[:, 1]] + v_feats[face[:, 2]]) / 3.0


def _f2v(f_feats, face, nv, nf_count):
    out = jnp.zeros((nv, f_feats.shape[1]), jnp.float32)
    out = out.at[face[:, 0]].add(f_feats).at[face[:, 1]].add(f_feats).at[face[:, 2]].add(f_feats)
    return out / jnp.maximum(nf_count[:, None], 1.0)


def _mesh_conv(v_feats, face, nv, nf_count, filt, W, b):
    ff = _v2f(v_feats, face)
    out = _spharm_conv(ff, filt, W) + b
    return jax.nn.relu(_f2v(out, face, nv, nf_count))


@functools.lru_cache(maxsize=1)
def _sc_mesh():
    return plsc.VectorSubcoreMesh(core_axis_name="c", subcore_axis_name="s")


def _gather_layout(e):
    cands = []
    for active in range(1, 17):
        gran = 2 * active * 128
        trips = -(-e // gran)
        cands.append((gran * trips, active, trips))
    mp_min = min(c[0] for c in cands)
    ep, active, trips = max((c for c in cands if c[0] <= mp_min * 1.10),
                            key=lambda c: c[1])
    return active, trips, ep


def _sc_gather(table, idx, active, trips):
    t_rows, cc = table.shape
    ep = idx.shape[0]

    def body(table_hbm, idx_hbm, out_hbm, idx_v, rows_v, sem):
        c = lax.axis_index("c")
        s = lax.axis_index("s")
        wid = s * 2 + c
        base = wid * (trips * 128)

        def chunk(t, carry):
            off = pl.multiple_of(base + t * 128, 8)
            pltpu.sync_copy(idx_hbm.at[pl.ds(off, 128)], idx_v)
            pltpu.async_copy(table_hbm.at[idx_v], rows_v, sem).wait()
            pltpu.sync_copy(rows_v, out_hbm.at[pl.ds(off, 128)])
            return carry

        @pl.when(wid < 2 * active)
        def _():
            lax.fori_loop(0, trips, chunk, 0)

    return pl.kernel(
        body,
        out_type=jax.ShapeDtypeStruct((ep, cc), jnp.float32),
        mesh=_sc_mesh(),
        compiler_params=pltpu.CompilerParams(use_tc_tiling_on_sc=False),
        scratch_types=[
            pltpu.VMEM((128,), jnp.int32),
            pltpu.VMEM((128, cc), jnp.float32),
            pltpu.SemaphoreType.DMA,
        ],
    )(table, idx)


def _pad_corner_idx(face, ep):
    m = face.shape[0]
    flat = face.T.reshape(3 * m).astype(jnp.int32)
    return jnp.zeros((ep,), jnp.int32).at[:3 * m].set(flat)


def _v2f_sc(v_feats, face):
    m = face.shape[0]
    active, trips, ep = _gather_layout(3 * m)
    idx = _pad_corner_idx(face, ep)
    rows = _sc_gather(v_feats, idx, active, trips)
    g = rows[:3 * m].reshape(3, m, v_feats.shape[1])
    return (g[0] + g[1] + g[2]) * (1.0 / 3.0)


def _corner_gather_sc(v_pad16, face):
    m = face.shape[0]
    active, trips, ep = _gather_layout(3 * m)
    idx = _pad_corner_idx(face, ep)
    rows = _sc_gather(v_pad16, idx, active, trips)
    return rows[:3 * m].reshape(3, m, 16)


def _matmul_kernel(x_ref, w_ref, b_ref, o_ref):
    o_ref[...] = jnp.dot(x_ref[...], w_ref[...],
                         preferred_element_type=jnp.float32) + b_ref[...]


def _matmul_relu_kernel(x_ref, w_ref, b_ref, o_ref):
    o_ref[...] = jax.nn.relu(jnp.dot(x_ref[...], w_ref[...],
                                     preferred_element_type=jnp.float32) + b_ref[...])


def _matmul_bias(x, W, b, block=1024, relu=False):
    n, cin = x.shape
    cout = W.shape[1]
    grid = (pl.cdiv(n, block),)
    return pl.pallas_call(
        _matmul_relu_kernel if relu else _matmul_kernel,
        grid=grid,
        in_specs=[
            pl.BlockSpec((block, cin), lambda i: (i, 0)),
            pl.BlockSpec((cin, cout), lambda i: (0, 0)),
            pl.BlockSpec((cout,), lambda i: (0,)),
        ],
        out_specs=pl.BlockSpec((block, cout), lambda i: (i, 0)),
        out_shape=jax.ShapeDtypeStruct((n, cout), jnp.float32),
    )(x, W, b)


def kernel(vertex_in, face_in, nv_in, mf_in, facet_textures, bary_coeff, num_texture, params):
    vertex = vertex_in[:, :3]
    nv0 = vertex.shape[0]

    vs = [vertex]
    fs = [face_in]
    nvs = [nv0]
    for s in STRIDE:
        nv = nvs[-1]
        nvo = max(nv // s, 1)
        vs.append(_seg_mean_pos(vs[-1], s))
        f_next = jnp.minimum(fs[-1] // s, nvo - 1)[::s]
        fs.append(f_next)
        nvs.append(nvo)

    f0 = fs[0]
    V1, V2, V3 = vertex[f0[:, 0]], vertex[f0[:, 1]], vertex[f0[:, 2]]
    n0 = _oriented_normals(V1, V2, V3)
    geom = _facet_geometry(V1, V2, V3, n0)
    filt0 = _sph_harm(n0)
    geo_face = jax.nn.relu(geom @ params['Wg'] + params['bg'])
    bary_sph = _sph_harm(bary_coeff)
    tex_face = jnp.einsum('nk,nc,kcg->ng', bary_sph, facet_textures, params['Wt'])
    denom = jnp.maximum(num_texture.astype(jnp.float32)[:, None], 1.0)
    tex_face = jax.nn.relu(tex_face / denom + params['bt0'])
    face_feat = geo_face + tex_face

    nfc0 = _count_adj(f0, nv0)
    feats = jax.nn.relu(_f2v(_spharm_conv(face_feat, filt0, params['Wf0'])
                             + params['bf0'], f0, nv0, nfc0))

    decoder_helper = [feats]
    feats = _seg_max(feats, STRIDE[0])

    for k in range(5):
        fk = fs[k + 1]
        nvk = nvs[k + 1]
        nfck = _count_adj(fk, nvk)
        Vk = vs[k + 1]
        nk = _oriented_normals(Vk[fk[:, 0]], Vk[fk[:, 1]], Vk[fk[:, 2]])
        filtk = _sph_harm(nk)
        for i in range(NITERS[k]):
            new = _mesh_conv(feats, fk, nvk, nfck, filtk,
                             params['B%d_W%d' % (k, i)], params['B%d_b%d' % (k, i)])
            feats = jnp.concatenate([feats, new], axis=-1)
        feats = _matmul_bias(feats, params['B%d_Wt' % k], params['B%d_bt' % k], relu=True)
        if k < 4:
            decoder_helper.append(feats)
            feats = _seg_max(feats, STRIDE[k + 1])

    for k in range(5):
        it = 5 - k
        low = decoder_helper[it - 1]
        up = _upsample(feats, STRIDE[it - 1], nvs[it - 1])
        feats = jnp.concatenate([low, up], axis=-1)
        feats = _matmul_bias(feats, params['D%d_W' % k], params['D%d_b' % k], relu=True)

    return _matmul_bias(feats, params['Wp'], params['bp'])


# speedup vs baseline: 1.5323x; 1.0854x over previous
import functools

import jax
import jax.numpy as jnp
from jax import lax
from jax.experimental import pallas as pl
from jax.experimental.pallas import tpu as pltpu
from jax.experimental.pallas import tpu_sc as plsc

L_SPH = 3
K = (L_SPH + 1) ** 2
NUM_CLASS = 20
STRIDE = [4, 3, 3, 2, 2]
ENC = [32, 64, 96, 128, 192, 256]
DEC = [256, 128, 128, 96, 96, 96]
NITERS = [2, 2, 4, 4, 4]
GROWTH = 32


def _norm(v):
    return jnp.sqrt(jnp.sum(v * v, axis=-1, keepdims=True) + 1e-12)


def _sph_harm(n):
    x, y, z = n[:, 0], n[:, 1], n[:, 2]
    one = jnp.ones_like(x)
    c = [0.28209479 * one, 0.48860251 * y, 0.48860251 * z, 0.48860251 * x,
         1.09254843 * x * y, 1.09254843 * y * z, 0.31539157 * (3.0 * z * z - 1.0),
         1.09254843 * x * z, 0.54627422 * (x * x - y * y),
         0.59004359 * y * (3.0 * x * x - y * y), 2.89061144 * x * y * z,
         0.45704580 * y * (5.0 * z * z - 1.0), 0.37317633 * z * (5.0 * z * z - 3.0),
         0.45704580 * x * (5.0 * z * z - 1.0), 1.44530572 * z * (x * x - y * y),
         0.59004359 * x * (x * x - 3.0 * y * y)]
    return jnp.stack(c, axis=1)


def _seg_mean_pos(v, s):
    nv = v.shape[0]
    nvo = max(nv // s, 1)
    main = v[: nvo * s].reshape(nvo, s, v.shape[1]).sum(axis=1)
    tail = v[nvo * s:]
    cnt = jnp.full((nvo, 1), float(s), jnp.float32)
    if tail.shape[0]:
        main = main.at[-1].add(tail.sum(axis=0))
        cnt = cnt.at[-1].add(float(tail.shape[0]))
    return main / cnt


def _seg_max(x, s):
    nv = x.shape[0]
    nvo = max(nv // s, 1)
    main = x[: nvo * s].reshape(nvo, s, x.shape[1]).max(axis=1)
    tail = x[nvo * s:]
    if tail.shape[0]:
        main = main.at[-1].max(tail.max(axis=0))
    return main


def _upsample(x, s, nv):
    nvo = x.shape[0]
    up = jnp.repeat(x, s, axis=0)
    if nv > nvo * s:
        up = jnp.concatenate(
            [up, jnp.broadcast_to(x[-1:], (nv - nvo * s, x.shape[1]))], axis=0)
    return up[:nv]


def _oriented_normals(V1, V2, V3):
    e1 = V2 - V1
    e2 = V3 - V1
    n = jnp.cross(e1, e2)
    n = n / _norm(n)
    center = (V1 + V2 + V3) / 3.0
    sign = jnp.sign(jnp.sum(-center * n, axis=-1, keepdims=True))
    sign = jnp.where(sign > -1.0, 1.0, sign)
    return n * sign


def _facet_geometry(V1, V2, V3, normals):
    D12 = V2 - V1
    D23 = V3 - V2
    D31 = V1 - V3
    L12 = _norm(D12)
    L23 = _norm(D23)
    L31 = _norm(D31)
    T1 = jnp.sum(D12 * (-D31), axis=-1, keepdims=True) / (L12 * L31)
    T2 = jnp.sum((-D12) * D23, axis=-1, keepdims=True) / (L12 * L23)
    T3 = jnp.sum((-D23) * D31, axis=-1, keepdims=True) / (L23 * L31)
    return jnp.concatenate([L12, L23, L31, T1, T2, T3, normals], axis=-1)


def _count_adj(face, nv):
    c = jnp.zeros((nv,), jnp.float32)
    return c.at[face[:, 0]].add(1.0).at[face[:, 1]].add(1.0).at[face[:, 2]].add(1.0)


def _spharm_conv(ff, filt, W):
    return jnp.einsum('nk,nc,kcg->ng', filt, ff, W)

def _spconv_body(ff_ref, filt_ref, w_ref, b_ref, o_ref):
    acc = jnp.broadcast_to(b_ref[...], o_ref.shape).astype(jnp.float32)
    ff = ff_ref[...]
    filt = filt_ref[...]
    for k in range(K):
        acc = acc + filt[:, k:k + 1] * jnp.dot(ff, w_ref[k],
                                               preferred_element_type=jnp.float32)
    o_ref[...] = acc


def _spharm_conv_b(ff, filt, W, b, block=2048):
    m, cin = ff.shape
    g = W.shape[2]
    grid = (pl.cdiv(m, block),)
    return pl.pallas_call(
        _spconv_body,
        grid=grid,
        in_specs=[
            pl.BlockSpec((block, cin), lambda i: (i, 0)),
            pl.BlockSpec((block, K), lambda i: (i, 0)),
            pl.BlockSpec((K, cin, g), lambda i: (0, 0, 0)),
            pl.BlockSpec((g,), lambda i: (0,)),
        ],
        out_specs=pl.BlockSpec((block, g), lambda i: (i, 0)),
        out_shape=jax.ShapeDtypeStruct((m, g), jnp.float32),
    )(ff, filt, W, b)


def _v2f(v_feats, face):
    return (v_feats[face[:, 0]] + v_feats[face[:, 1]] + v_feats[face[:, 2]]) / 3.0


def _f2v(f_feats, face, nv, nf_count):
    out = jnp.zeros((nv, f_feats.shape[1]), jnp.float32)
    out = out.at[face[:, 0]].add(f_feats).at[face[:, 1]].add(f_feats).at[face[:, 2]].add(f_feats)
    return out / jnp.maximum(nf_count[:, None], 1.0)


def _mesh_conv(v_feats, face, nv, nf_count, filt, W, b):
    ff = _v2f(v_feats, face)
    out = _spharm_conv(ff, filt, W) + b
    return jax.nn.relu(_f2v(out, face, nv, nf_count))


@functools.lru_cache(maxsize=1)
def _sc_mesh():
    return plsc.VectorSubcoreMesh(core_axis_name="c", subcore_axis_name="s")


def _gather_layout(e):
    cands = []
    for active in range(1, 17):
        gran = 2 * active * 128
        trips = -(-e // gran)
        cands.append((gran * trips, active, trips))
    mp_min = min(c[0] for c in cands)
    ep, active, trips = max((c for c in cands if c[0] <= mp_min * 1.10),
                            key=lambda c: c[1])
    return active, trips, ep


def _sc_gather(table, idx, active, trips):
    t_rows, cc = table.shape
    ep = idx.shape[0]

    def body(table_hbm, idx_hbm, out_hbm, idx_v, rows_v, sem):
        c = lax.axis_index("c")
        s = lax.axis_index("s")
        wid = s * 2 + c
        base = wid * (trips * 128)

        def chunk(t, carry):
            off = pl.multiple_of(base + t * 128, 8)
            pltpu.sync_copy(idx_hbm.at[pl.ds(off, 128)], idx_v)
            pltpu.async_copy(table_hbm.at[idx_v], rows_v, sem).wait()
            pltpu.sync_copy(rows_v, out_hbm.at[pl.ds(off, 128)])
            return carry

        @pl.when(wid < 2 * active)
        def _():
            lax.fori_loop(0, trips, chunk, 0)

    return pl.kernel(
        body,
        out_type=jax.ShapeDtypeStruct((ep, cc), jnp.float32),
        mesh=_sc_mesh(),
        compiler_params=pltpu.CompilerParams(use_tc_tiling_on_sc=False),
        scratch_types=[
            pltpu.VMEM((128,), jnp.int32),
            pltpu.VMEM((128, cc), jnp.float32),
            pltpu.SemaphoreType.DMA,
        ],
    )(table, idx)


def _pad_corner_idx(face, ep):
    m = face.shape[0]
    flat = face.T.reshape(3 * m).astype(jnp.int32)
    return jnp.zeros((ep,), jnp.int32).at[:3 * m].set(flat)


def _v2f_sc(v_feats, face):
    m = face.shape[0]
    active, trips, ep = _gather_layout(3 * m)
    idx = _pad_corner_idx(face, ep)
    rows = _sc_gather(v_feats, idx, active, trips)
    g = rows[:3 * m].reshape(3, m, v_feats.shape[1])
    return (g[0] + g[1] + g[2]) * (1.0 / 3.0)


def _corner_gather_sc(v_pad16, face):
    m = face.shape[0]
    active, trips, ep = _gather_layout(3 * m)
    idx = _pad_corner_idx(face, ep)
    rows = _sc_gather(v_pad16, idx, active, trips)
    return rows[:3 * m].reshape(3, m, 16)


def _matmul_kernel(x_ref, w_ref, b_ref, o_ref):
    o_ref[...] = jnp.dot(x_ref[...], w_ref[...],
                         preferred_element_type=jnp.float32) + b_ref[...]


def _matmul_relu_kernel(x_ref, w_ref, b_ref, o_ref):
    o_ref[...] = jax.nn.relu(jnp.dot(x_ref[...], w_ref[...],
                                     preferred_element_type=jnp.float32) + b_ref[...])


def _matmul_bias(x, W, b, block=1024, relu=False):
    n, cin = x.shape
    cout = W.shape[1]
    grid = (pl.cdiv(n, block),)
    return pl.pallas_call(
        _matmul_relu_kernel if relu else _matmul_kernel,
        grid=grid,
        in_specs=[
            pl.BlockSpec((block, cin), lambda i: (i, 0)),
            pl.BlockSpec((cin, cout), lambda i: (0, 0)),
            pl.BlockSpec((cout,), lambda i: (0,)),
        ],
        out_specs=pl.BlockSpec((block, cout), lambda i: (i, 0)),
        out_shape=jax.ShapeDtypeStruct((n, cout), jnp.float32),
    )(x, W, b)


def kernel(vertex_in, face_in, nv_in, mf_in, facet_textures, bary_coeff, num_texture, params):
    vertex = vertex_in[:, :3]
    nv0 = vertex.shape[0]

    vs = [vertex]
    fs = [face_in]
    nvs = [nv0]
    for s in STRIDE:
        nv = nvs[-1]
        nvo = max(nv // s, 1)
        vs.append(_seg_mean_pos(vs[-1], s))
        f_next = jnp.minimum(fs[-1] // s, nvo - 1)[::s]
        fs.append(f_next)
        nvs.append(nvo)

    f0 = fs[0]
    v0p = jnp.zeros((nv0, 16), jnp.float32).at[:, :3].set(vertex)
    C0 = _corner_gather_sc(v0p, f0)
    V1, V2, V3 = C0[0, :, :3], C0[1, :, :3], C0[2, :, :3]
    n0 = _oriented_normals(V1, V2, V3)
    geom = _facet_geometry(V1, V2, V3, n0)
    filt0 = _sph_harm(n0)
    geo_face = jax.nn.relu(geom @ params['Wg'] + params['bg'])
    bary_sph = _sph_harm(bary_coeff)
    tex_face = jnp.einsum('nk,nc,kcg->ng', bary_sph, facet_textures, params['Wt'])
    denom = jnp.maximum(num_texture.astype(jnp.float32)[:, None], 1.0)
    tex_face = jax.nn.relu(tex_face / denom + params['bt0'])
    face_feat = geo_face + tex_face

    nfc0 = _count_adj(f0, nv0)
    feats = jax.nn.relu(_f2v(_spharm_conv(face_feat, filt0, params['Wf0'])
                             + params['bf0'], f0, nv0, nfc0))

    decoder_helper = [feats]
    feats = _seg_max(feats, STRIDE[0])

    for k in range(5):
        fk = fs[k + 1]
        nvk = nvs[k + 1]
        nfck = _count_adj(fk, nvk)
        Vk = vs[k + 1]
        nk = _oriented_normals(Vk[fk[:, 0]], Vk[fk[:, 1]], Vk[fk[:, 2]])
        filtk = _sph_harm(nk)
        for i in range(NITERS[k]):
            new = _mesh_conv(feats, fk, nvk, nfck, filtk,
                             params['B%d_W%d' % (k, i)], params['B%d_b%d' % (k, i)])
            feats = jnp.concatenate([feats, new], axis=-1)
        feats = _matmul_bias(feats, params['B%d_Wt' % k], params['B%d_bt' % k], relu=True)
        if k < 4:
            decoder_helper.append(feats)
            feats = _seg_max(feats, STRIDE[k + 1])

    for k in range(5):
        it = 5 - k
        low = decoder_helper[it - 1]
        up = _upsample(feats, STRIDE[it - 1], nvs[it - 1])
        feats = jnp.concatenate([low, up], axis=-1)
        feats = _matmul_bias(feats, params['D%d_W' % k], params['D%d_b' % k], relu=True)

    return _matmul_bias(feats, params['Wp'], params['bp'])
